# baseline (device time: 49609 ns/iter reference)
import jax
import jax.numpy as jnp
from jax import lax
from jax.experimental import pallas as pl
from jax.experimental.pallas import tpu as pltpu

N_DEV = 4


def kernel(x, W1, W2):
    m_per, d = x.shape
    mh = m_per // 2

    def body(x_ref, w1_ref, w2_ref, out_ref,
             w1v, w2v, outv,
             xgA, accA, stgA, xgB, accB, stgB,
             cp_sems,
             agA_s, agA_r, rsA_s, rsA_r,
             agB_s, agB_r, rsB_s, rsB_r):
        my = lax.axis_index("i")
        left = (my - 1) % N_DEV
        right = (my + 1) % N_DEV

        cpxA = pltpu.make_async_copy(x_ref.at[0:mh, :], xgA.at[0], cp_sems.at[0])
        cpxB = pltpu.make_async_copy(x_ref.at[mh:, :], xgB.at[0], cp_sems.at[1])
        cpw1 = pltpu.make_async_copy(w1_ref, w1v, cp_sems.at[2])
        cpw2 = pltpu.make_async_copy(w2_ref, w2v, cp_sems.at[3])
        cpxA.start()
        cpxB.start()
        cpw1.start()
        cpw2.start()

        barrier = pltpu.get_barrier_semaphore()
        for nbr in (left, right):
            pl.semaphore_signal(barrier, inc=1, device_id=(nbr,),
                                device_id_type=pl.DeviceIdType.MESH)
        pl.semaphore_wait(barrier, 2)

        def mk(src, dst, ssem, rsem, dev):
            return pltpu.make_async_remote_copy(
                src_ref=src, dst_ref=dst, send_sem=ssem, recv_sem=rsem,
                device_id=(dev,), device_id_type=pl.DeviceIdType.MESH)

        agA = [mk(xgA.at[h], xgA.at[h + 1], agA_s.at[h], agA_r.at[h], right)
               for h in range(N_DEV - 1)]
        agB = [mk(xgB.at[h], xgB.at[h + 1], agB_s.at[h], agB_r.at[h], left)
               for h in range(N_DEV - 1)]
        rsA = [mk(accA.at[s], stgA.at[s], rsA_s.at[s], rsA_r.at[s], right)
               for s in range(N_DEV - 1)]
        rsB = [mk(accB.at[s], stgB.at[s], rsB_s.at[s], rsB_r.at[s], left)
               for s in range(N_DEV - 1)]

        def partial(xb):
            h1 = jnp.dot(xb, w1v[...], preferred_element_type=jnp.float32)
            h1 = h1 * jax.nn.sigmoid(h1)
            return jnp.dot(h1, w2v[...], preferred_element_type=jnp.float32)

        cpxA.wait()
        cpxB.wait()
        agA[0].start()
        agB[0].start()

        cpw1.wait()
        cpw2.wait()
        accA[3] = partial(xgA[0])
        accB[3] = partial(xgB[0])

        agA[0].wait_recv()
        agA[1].start()
        agB[0].wait_recv()
        agB[1].start()
        accA[0] = partial(xgA[1])
        accB[0] = partial(xgB[1])
        rsA[0].start()
        rsB[0].start()

        agA[1].wait_recv()
        agA[2].start()
        agB[1].wait_recv()
        agB[2].start()
        accA[1] = partial(xgA[2])
        accB[1] = partial(xgB[2])

        rsA[0].wait_recv()
        accA[1] = accA[1] + stgA[0]
        rsA[1].start()
        rsB[0].wait_recv()
        accB[1] = accB[1] + stgB[0]
        rsB[1].start()

        agA[2].wait_recv()
        accA[2] = partial(xgA[3])
        agB[2].wait_recv()
        accB[2] = partial(xgB[3])

        rsA[1].wait_recv()
        accA[2] = accA[2] + stgA[1]
        rsA[2].start()
        rsB[1].wait_recv()
        accB[2] = accB[2] + stgB[1]
        rsB[2].start()

        rsA[2].wait_recv()
        outv[0:mh, :] = accA[3] + stgA[2]
        cpoA = pltpu.make_async_copy(outv.at[0:mh, :], out_ref.at[0:mh, :],
                                     cp_sems.at[0])
        cpoA.start()
        rsB[2].wait_recv()
        outv[mh:, :] = accB[3] + stgB[2]
        cpoB = pltpu.make_async_copy(outv.at[mh:, :], out_ref.at[mh:, :],
                                     cp_sems.at[1])
        cpoB.start()
        cpoA.wait()
        cpoB.wait()

        for r in agA + agB + rsA + rsB:
            r.wait_send()

    half = pltpu.VMEM((N_DEV, mh, d), jnp.float32)
    stage = pltpu.VMEM((N_DEV - 1, mh, d), jnp.float32)
    sems = pltpu.SemaphoreType.DMA((N_DEV - 1,))
    return pl.pallas_call(
        body,
        out_shape=jax.ShapeDtypeStruct((m_per, d), jnp.float32),
        in_specs=[pl.BlockSpec(memory_space=pl.ANY)] * 3,
        out_specs=pl.BlockSpec(memory_space=pl.ANY),
        scratch_shapes=[
            pltpu.VMEM(W1.shape, jnp.float32),
            pltpu.VMEM(W2.shape, jnp.float32),
            pltpu.VMEM((m_per, d), jnp.float32),
            half, half, stage,
            half, half, stage,
            pltpu.SemaphoreType.DMA((4,)),
            sems, sems, sems, sems,
            sems, sems, sems, sems,
        ],
        compiler_params=pltpu.CompilerParams(collective_id=0),
    )(x, W1, W2)


# device time: 32351 ns/iter; 1.5335x vs baseline; 1.5335x over previous
import jax
import jax.numpy as jnp
from jax import lax
from jax.experimental import pallas as pl
from jax.experimental.pallas import tpu as pltpu

N_DEV = 4


def kernel(x, W1, W2):
    m_per, d = x.shape
    mh = m_per // 2

    def body(x_ref, w1_ref, w2_ref, out_ref,
             xgA, accA, stgA, xgB, accB, stgB,
             agA_s, agA_r, rsA_s, rsA_r,
             agB_s, agB_r, rsB_s, rsB_r):
        my = lax.axis_index("i")
        left = (my - 1) % N_DEV
        right = (my + 1) % N_DEV

        barrier = pltpu.get_barrier_semaphore()
        for nbr in (left, right):
            pl.semaphore_signal(barrier, inc=1, device_id=(nbr,),
                                device_id_type=pl.DeviceIdType.MESH)
        pl.semaphore_wait(barrier, 2)

        def mk(src, dst, ssem, rsem, dev):
            return pltpu.make_async_remote_copy(
                src_ref=src, dst_ref=dst, send_sem=ssem, recv_sem=rsem,
                device_id=(dev,), device_id_type=pl.DeviceIdType.MESH)

        agA = [mk(xgA.at[h], xgA.at[h + 1], agA_s.at[h], agA_r.at[h], right)
               for h in range(N_DEV - 1)]
        agB = [mk(xgB.at[h], xgB.at[h + 1], agB_s.at[h], agB_r.at[h], left)
               for h in range(N_DEV - 1)]
        rsA = [mk(accA.at[s], stgA.at[s], rsA_s.at[s], rsA_r.at[s], right)
               for s in range(N_DEV - 1)]
        rsB = [mk(accB.at[s], stgB.at[s], rsB_s.at[s], rsB_r.at[s], left)
               for s in range(N_DEV - 1)]

        def partial(xb):
            h1 = jnp.dot(xb.astype(jnp.float32), w1_ref[...],
                         preferred_element_type=jnp.float32)
            h1 = h1 * jax.nn.sigmoid(h1)
            return jnp.dot(h1, w2_ref[...],
                           preferred_element_type=jnp.float32)

        xgA[0] = x_ref[0:mh, :].astype(jnp.bfloat16)
        xgB[0] = x_ref[mh:, :].astype(jnp.bfloat16)
        agA[0].start()
        agB[0].start()

        accA[3] = partial(xgA[0]).astype(jnp.bfloat16)
        accB[3] = partial(xgB[0]).astype(jnp.bfloat16)

        agA[0].wait_recv()
        agA[1].start()
        agB[0].wait_recv()
        agB[1].start()
        accA[0] = partial(xgA[1]).astype(jnp.bfloat16)
        accB[0] = partial(xgB[1]).astype(jnp.bfloat16)
        rsA[0].start()
        rsB[0].start()

        agA[1].wait_recv()
        agA[2].start()
        agB[1].wait_recv()
        agB[2].start()
        accA[1] = partial(xgA[2]).astype(jnp.bfloat16)
        accB[1] = partial(xgB[2]).astype(jnp.bfloat16)

        rsA[0].wait_recv()
        accA[1] = accA[1] + stgA[0]
        rsA[1].start()
        rsB[0].wait_recv()
        accB[1] = accB[1] + stgB[0]
        rsB[1].start()

        agA[2].wait_recv()
        accA[2] = partial(xgA[3]).astype(jnp.bfloat16)
        agB[2].wait_recv()
        accB[2] = partial(xgB[3]).astype(jnp.bfloat16)

        rsA[1].wait_recv()
        accA[2] = accA[2] + stgA[1]
        rsA[2].start()
        rsB[1].wait_recv()
        accB[2] = accB[2] + stgB[1]
        rsB[2].start()

        rsA[2].wait_recv()
        out_ref[0:mh, :] = (accA[3].astype(jnp.float32)
                            + stgA[2].astype(jnp.float32))
        rsB[2].wait_recv()
        out_ref[mh:, :] = (accB[3].astype(jnp.float32)
                           + stgB[2].astype(jnp.float32))

        for r in agA + agB + rsA + rsB:
            r.wait_send()

    half = pltpu.VMEM((N_DEV, mh, d), jnp.bfloat16)
    stage = pltpu.VMEM((N_DEV - 1, mh, d), jnp.bfloat16)
    sems = pltpu.SemaphoreType.DMA((N_DEV - 1,))
    return pl.pallas_call(
        body,
        out_shape=jax.ShapeDtypeStruct((m_per, d), jnp.float32),
        in_specs=[pl.BlockSpec(memory_space=pltpu.VMEM)] * 3,
        out_specs=pl.BlockSpec(memory_space=pltpu.VMEM),
        scratch_shapes=[
            half, half, stage,
            half, half, stage,
            sems, sems, sems, sems,
            sems, sems, sems, sems,
        ],
        compiler_params=pltpu.CompilerParams(collective_id=0),
    )(x, W1, W2)


# device time: 32049 ns/iter; 1.5479x vs baseline; 1.0094x over previous
import jax
import jax.numpy as jnp
from jax import lax
from jax.experimental import pallas as pl
from jax.experimental.pallas import tpu as pltpu

N_DEV = 4


def kernel(x, W1, W2):
    m_per, d = x.shape
    mh = m_per // 2

    def body(x_ref, w1_ref, w2_ref, out_ref,
             w1b, w2b,
             xgA, accA, stgA, xgB, accB, stgB,
             agA_s, agA_r, rsA_s, rsA_r,
             agB_s, agB_r, rsB_s, rsB_r):
        my = lax.axis_index("i")
        left = (my - 1) % N_DEV
        right = (my + 1) % N_DEV

        barrier = pltpu.get_barrier_semaphore()
        for nbr in (left, right):
            pl.semaphore_signal(barrier, inc=1, device_id=(nbr,),
                                device_id_type=pl.DeviceIdType.MESH)
        pl.semaphore_wait(barrier, 2)

        def mk(src, dst, ssem, rsem, dev):
            return pltpu.make_async_remote_copy(
                src_ref=src, dst_ref=dst, send_sem=ssem, recv_sem=rsem,
                device_id=(dev,), device_id_type=pl.DeviceIdType.MESH)

        agA = [mk(xgA.at[h], xgA.at[h + 1], agA_s.at[h], agA_r.at[h], right)
               for h in range(N_DEV - 1)]
        agB = [mk(xgB.at[h], xgB.at[h + 1], agB_s.at[h], agB_r.at[h], left)
               for h in range(N_DEV - 1)]
        rsA = [mk(accA.at[s], stgA.at[s], rsA_s.at[s], rsA_r.at[s], right)
               for s in range(N_DEV - 1)]
        rsB = [mk(accB.at[s], stgB.at[s], rsB_s.at[s], rsB_r.at[s], left)
               for s in range(N_DEV - 1)]

        def partial(xb):
            h1 = jnp.dot(xb, w1b[...], preferred_element_type=jnp.float32)
            h1 = h1 * jax.nn.sigmoid(h1)
            return jnp.dot(h1.astype(jnp.bfloat16), w2b[...],
                           preferred_element_type=jnp.float32)

        xgA[0] = x_ref[0:mh, :].astype(jnp.bfloat16)
        xgB[0] = x_ref[mh:, :].astype(jnp.bfloat16)
        agA[0].start()
        agB[0].start()

        w1b[...] = w1_ref[...].astype(jnp.bfloat16)
        w2b[...] = w2_ref[...].astype(jnp.bfloat16)

        accA[3] = partial(xgA[0]).astype(jnp.bfloat16)
        accB[3] = partial(xgB[0]).astype(jnp.bfloat16)

        agA[0].wait_recv()
        agA[1].start()
        agB[0].wait_recv()
        agB[1].start()
        accA[0] = partial(xgA[1]).astype(jnp.bfloat16)
        accB[0] = partial(xgB[1]).astype(jnp.bfloat16)
        rsA[0].start()
        rsB[0].start()

        agA[1].wait_recv()
        agA[2].start()
        agB[1].wait_recv()
        agB[2].start()
        accA[1] = partial(xgA[2]).astype(jnp.bfloat16)
        accB[1] = partial(xgB[2]).astype(jnp.bfloat16)

        rsA[0].wait_recv()
        accA[1] = accA[1] + stgA[0]
        rsA[1].start()
        rsB[0].wait_recv()
        accB[1] = accB[1] + stgB[0]
        rsB[1].start()

        agA[2].wait_recv()
        accA[2] = partial(xgA[3]).astype(jnp.bfloat16)
        agB[2].wait_recv()
        accB[2] = partial(xgB[3]).astype(jnp.bfloat16)

        rsA[1].wait_recv()
        accA[2] = accA[2] + stgA[1]
        rsA[2].start()
        rsB[1].wait_recv()
        accB[2] = accB[2] + stgB[1]
        rsB[2].start()

        rsA[2].wait_recv()
        out_ref[0:mh, :] = (accA[3].astype(jnp.float32)
                            + stgA[2].astype(jnp.float32))
        rsB[2].wait_recv()
        out_ref[mh:, :] = (accB[3].astype(jnp.float32)
                           + stgB[2].astype(jnp.float32))

        for r in agA + agB + rsA + rsB:
            r.wait_send()

    half = pltpu.VMEM((N_DEV, mh, d), jnp.bfloat16)
    stage = pltpu.VMEM((N_DEV - 1, mh, d), jnp.bfloat16)
    sems = pltpu.SemaphoreType.DMA((N_DEV - 1,))
    return pl.pallas_call(
        body,
        out_shape=jax.ShapeDtypeStruct((m_per, d), jnp.float32),
        in_specs=[pl.BlockSpec(memory_space=pltpu.VMEM)] * 3,
        out_specs=pl.BlockSpec(memory_space=pltpu.VMEM),
        scratch_shapes=[
            pltpu.VMEM(W1.shape, jnp.bfloat16),
            pltpu.VMEM(W2.shape, jnp.bfloat16),
            half, half, stage,
            half, half, stage,
            sems, sems, sems, sems,
            sems, sems, sems, sems,
        ],
        compiler_params=pltpu.CompilerParams(collective_id=0),
    )(x, W1, W2)
